# Initial kernel scaffold; baseline (speedup 1.0000x reference)
import jax
import jax.numpy as jnp
from jax.experimental import pallas as pl


def kernel(x, adj_t, W1, b1, W2, b2, W3, b3):
    raise NotImplementedError("write your pallas kernel here")


# speedup vs baseline: 6.8520x; 6.8520x over previous
import functools

import jax
import jax.numpy as jnp
from jax import lax
from jax.experimental import pallas as pl
from jax.experimental.pallas import tpu as pltpu
from jax.experimental.pallas import tpu_sc as plsc

_NC = 2
_NS = 16


def _sc_mesh():
    return plsc.VectorSubcoreMesh(
        core_axis_name="c", subcore_axis_name="s",
        num_cores=_NC, num_subcores=_NS)


def _pad(n):
    rps = -(-n // _NS)
    rps = -(-rps // 128) * 128
    return rps, rps * _NS


def _sc_degree(dst, n):
    e = dst.shape[0]
    per_w = e // (_NC * _NS)
    blk = 40
    steps = per_w // blk
    rps, n_pad = _pad(n)

    @functools.partial(
        pl.kernel,
        out_type=jax.ShapeDtypeStruct((_NC, n_pad, 128), jnp.float32),
        mesh=_sc_mesh(),
        scratch_types=[
            pltpu.VMEM((blk,), jnp.int32),
            pltpu.VMEM((blk, 128), jnp.float32),
            pltpu.VMEM((128, 128), jnp.float32),
            pltpu.VMEM_SHARED((n_pad, 128), jnp.float32),
        ],
    )
    def kern(dst_hbm, out_hbm, idx_v, ones_v, zero_v, acc):
        c = lax.axis_index("c")
        s = lax.axis_index("s")
        w = s * _NC + c

        @pl.loop(0, blk)
        def _(r):
            @pl.loop(0, 8)
            def _(g):
                ones_v[r, pl.ds(g * 16, 16)] = jnp.ones((16,), jnp.float32)

        @pl.loop(0, 128)
        def _(r):
            @pl.loop(0, 8)
            def _(g):
                zero_v[r, pl.ds(g * 16, 16)] = jnp.zeros((16,), jnp.float32)

        @pl.loop(0, rps // 128)
        def _(k2):
            pltpu.sync_copy(zero_v, acc.at[pl.ds(s * rps + k2 * 128, 128)])

        plsc.subcore_barrier()

        base = w * per_w

        @pl.loop(0, steps)
        def _(i):
            pltpu.sync_copy(dst_hbm.at[pl.ds(base + i * blk, blk)], idx_v)
            pltpu.sync_copy(ones_v, acc.at[idx_v], add=True)

        plsc.subcore_barrier()
        pltpu.sync_copy(acc.at[pl.ds(s * rps, rps)],
                        out_hbm.at[c, pl.ds(s * rps, rps)])

    return kern(dst)


def _sc_agg_colsplit(hsplit, src, dst, n):
    e = src.shape[0]
    dh = hsplit.shape[1]
    per_s = e // _NS
    blk = 80
    steps = per_s // blk
    rps, n_pad = _pad(n)
    zr = 128

    @functools.partial(
        pl.kernel,
        out_type=jax.ShapeDtypeStruct((_NC, n_pad, dh), jnp.float32),
        mesh=_sc_mesh(),
        scratch_types=[
            pltpu.VMEM((blk,), jnp.int32),
            pltpu.VMEM((blk,), jnp.int32),
            pltpu.VMEM((blk,), jnp.int32),
            pltpu.VMEM((blk, dh), jnp.float32),
            pltpu.VMEM((zr, dh), jnp.float32),
            pltpu.VMEM_SHARED((n_pad, dh), jnp.float32),
        ],
    )
    def kern(h_hbm, src_hbm, dst_hbm, out_hbm, srcv, gsrcv, dstv, rows,
             zero_v, acc):
        c = lax.axis_index("c")
        s = lax.axis_index("s")
        coff = c * n

        @pl.loop(0, zr)
        def _(r):
            @pl.loop(0, dh // 16)
            def _(g):
                zero_v[r, pl.ds(g * 16, 16)] = jnp.zeros((16,), jnp.float32)

        @pl.loop(0, rps // zr)
        def _(k2):
            pltpu.sync_copy(zero_v, acc.at[pl.ds(s * rps + k2 * zr, zr)])

        plsc.subcore_barrier()

        base = s * per_s

        @pl.loop(0, steps)
        def _(i):
            pltpu.sync_copy(src_hbm.at[pl.ds(base + i * blk, blk)], srcv)
            pltpu.sync_copy(dst_hbm.at[pl.ds(base + i * blk, blk)], dstv)

            @pl.loop(0, blk // 16)
            def _(g):
                gsrcv[pl.ds(g * 16, 16)] = srcv[pl.ds(g * 16, 16)] + coff

            pltpu.sync_copy(h_hbm.at[gsrcv], rows)
            pltpu.sync_copy(rows, acc.at[dstv], add=True)

        plsc.subcore_barrier()
        pltpu.sync_copy(acc.at[pl.ds(s * rps, rps)],
                        out_hbm.at[c, pl.ds(s * rps, rps)])

    return kern(hsplit, src, dst)


def _sc_agg_edgesplit(h, src, dst, n):
    e = src.shape[0]
    dh = h.shape[1]
    per_w = e // (_NC * _NS)
    blk = 40
    steps = per_w // blk
    rps, n_pad = _pad(n)
    zr = 128

    @functools.partial(
        pl.kernel,
        out_type=jax.ShapeDtypeStruct((_NC, n_pad, dh), jnp.float32),
        mesh=_sc_mesh(),
        scratch_types=[
            pltpu.VMEM((blk,), jnp.int32),
            pltpu.VMEM((blk,), jnp.int32),
            pltpu.VMEM((blk, dh), jnp.float32),
            pltpu.VMEM((zr, dh), jnp.float32),
            pltpu.VMEM_SHARED((n_pad, dh), jnp.float32),
        ],
    )
    def kern(h_hbm, src_hbm, dst_hbm, out_hbm, srcv, dstv, rows, zero_v, acc):
        c = lax.axis_index("c")
        s = lax.axis_index("s")
        w = s * _NC + c

        @pl.loop(0, zr)
        def _(r):
            @pl.loop(0, dh // 16)
            def _(g):
                zero_v[r, pl.ds(g * 16, 16)] = jnp.zeros((16,), jnp.float32)

        @pl.loop(0, rps // zr)
        def _(k2):
            pltpu.sync_copy(zero_v, acc.at[pl.ds(s * rps + k2 * zr, zr)])

        plsc.subcore_barrier()

        base = w * per_w

        @pl.loop(0, steps)
        def _(i):
            pltpu.sync_copy(src_hbm.at[pl.ds(base + i * blk, blk)], srcv)
            pltpu.sync_copy(dst_hbm.at[pl.ds(base + i * blk, blk)], dstv)
            pltpu.sync_copy(h_hbm.at[srcv], rows)
            pltpu.sync_copy(rows, acc.at[dstv], add=True)

        plsc.subcore_barrier()
        pltpu.sync_copy(acc.at[pl.ds(s * rps, rps)],
                        out_hbm.at[c, pl.ds(s * rps, rps)])

    return kern(h, src, dst)


_BM = 1000


def _tc_first(x, W1, degp):
    n, d_in = x.shape
    d_out = W1.shape[1]
    dh = d_out // 2

    def body(x_ref, w_ref, d_ref, dis_ref, o_ref):
        deg = 1.0 + d_ref[0][:, 0:1] + d_ref[1][:, 0:1]
        dis = lax.rsqrt(deg)
        h = jnp.dot(x_ref[...], w_ref[...], preferred_element_type=jnp.float32)
        hp = h * dis
        dis_ref[...] = dis
        o_ref[0] = hp[:, :dh]
        o_ref[1] = hp[:, dh:]

    return pl.pallas_call(
        body,
        grid=(n // _BM,),
        in_specs=[
            pl.BlockSpec((_BM, d_in), lambda i: (i, 0)),
            pl.BlockSpec((d_in, d_out), lambda i: (0, 0)),
            pl.BlockSpec((2, _BM, 128), lambda i: (0, i, 0)),
        ],
        out_specs=[
            pl.BlockSpec((_BM, 1), lambda i: (i, 0)),
            pl.BlockSpec((2, _BM, dh), lambda i: (0, i, 0)),
        ],
        out_shape=[
            jax.ShapeDtypeStruct((n, 1), jnp.float32),
            jax.ShapeDtypeStruct((2, n, dh), jnp.float32),
        ],
    )(x, W1, degp)


def _tc_mid(aggp, hp, dis, b, W, split_out):
    n = hp.shape[1]
    dh = hp.shape[2]
    d_out = W.shape[1]
    wsplit = W.reshape(2, dh, d_out)
    b2 = b.reshape(2, 1, dh)

    def body(a_ref, h_ref, dis_ref, b_ref, w_ref, o_ref):
        dis = dis_ref[...]
        t = (a_ref[...] + h_ref[...]) * dis[None, :, :] + b_ref[...]
        t = jnp.maximum(t, 0.0)
        h = (jnp.dot(t[0], w_ref[0], preferred_element_type=jnp.float32)
             + jnp.dot(t[1], w_ref[1], preferred_element_type=jnp.float32))
        h = h * dis
        if split_out:
            o_ref[0] = h[:, :d_out // 2]
            o_ref[1] = h[:, d_out // 2:]
        else:
            o_ref[:, :d_out] = h
            o_ref[:, d_out:] = jnp.zeros((h.shape[0], 128 - d_out), jnp.float32)

    if split_out:
        out_spec = pl.BlockSpec((2, _BM, d_out // 2), lambda i: (0, i, 0))
        out_shape = jax.ShapeDtypeStruct((2, n, d_out // 2), jnp.float32)
    else:
        out_spec = pl.BlockSpec((_BM, 128), lambda i: (i, 0))
        out_shape = jax.ShapeDtypeStruct((n, 128), jnp.float32)

    return pl.pallas_call(
        body,
        grid=(n // _BM,),
        in_specs=[
            pl.BlockSpec((2, _BM, dh), lambda i: (0, i, 0)),
            pl.BlockSpec((2, _BM, dh), lambda i: (0, i, 0)),
            pl.BlockSpec((_BM, 1), lambda i: (i, 0)),
            pl.BlockSpec((2, 1, dh), lambda i: (0, 0, 0)),
            pl.BlockSpec((2, dh, d_out), lambda i: (0, 0, 0)),
        ],
        out_specs=out_spec,
        out_shape=out_shape,
    )(aggp, hp, dis, b2, wsplit)


def _tc_final(aggp, h3p, dis, b, d):
    n = h3p.shape[0]
    b2 = b.reshape(1, d)

    def body(a_ref, h_ref, dis_ref, b_ref, o_ref):
        t = a_ref[0] + a_ref[1] + h_ref[...]
        t = t[:, :d] * dis_ref[...] + b_ref[...]
        m = jnp.max(t, axis=1, keepdims=True)
        ex = jnp.exp(t - m)
        lse = jnp.log(jnp.sum(ex, axis=1, keepdims=True))
        o_ref[...] = t - m - lse

    return pl.pallas_call(
        body,
        grid=(n // _BM,),
        in_specs=[
            pl.BlockSpec((2, _BM, 128), lambda i: (0, i, 0)),
            pl.BlockSpec((_BM, 128), lambda i: (i, 0)),
            pl.BlockSpec((_BM, 1), lambda i: (i, 0)),
            pl.BlockSpec((1, d), lambda i: (0, 0)),
        ],
        out_specs=pl.BlockSpec((_BM, d), lambda i: (i, 0)),
        out_shape=jax.ShapeDtypeStruct((n, d), jnp.float32),
    )(aggp, h3p, dis, b2)


def kernel(x, adj_t, W1, b1, W2, b2, W3, b3):
    n = x.shape[0]
    src = adj_t[0]
    dst = adj_t[1]

    degp = _sc_degree(dst, n)
    dis, h1p = _tc_first(x, W1, degp)
    agg1 = _sc_agg_colsplit(h1p.reshape(2 * n, -1), src, dst, n)
    h2p = _tc_mid(agg1, h1p, dis, b1, W2, split_out=True)
    agg2 = _sc_agg_colsplit(h2p.reshape(2 * n, -1), src, dst, n)
    h3p = _tc_mid(agg2, h2p, dis, b2, W3, split_out=False)
    agg3 = _sc_agg_edgesplit(h3p, src, dst, n)
    return _tc_final(agg3, h3p, dis, b3, W3.shape[1])


# speedup vs baseline: 13.5356x; 1.9754x over previous
import functools

import jax
import jax.numpy as jnp
from jax import lax
from jax.experimental import pallas as pl
from jax.experimental.pallas import tpu as pltpu
from jax.experimental.pallas import tpu_sc as plsc

_NC = 2
_NS = 16


def _sc_mesh():
    return plsc.VectorSubcoreMesh(
        core_axis_name="c", subcore_axis_name="s",
        num_cores=_NC, num_subcores=_NS)


def _pad(n):
    rps = -(-n // _NS)
    rps = -(-rps // 128) * 128
    return rps, rps * _NS


def _sc_degree(dst, n):
    e = dst.shape[0]
    per_w = e // (_NC * _NS)
    blk = 40
    steps = per_w // blk
    rps, n_pad = _pad(n)

    @functools.partial(
        pl.kernel,
        out_type=jax.ShapeDtypeStruct((_NC, n_pad, 16), jnp.float32),
        mesh=_sc_mesh(),
        scratch_types=[
            pltpu.VMEM((blk,), jnp.int32),
            pltpu.VMEM((blk,), jnp.int32),
            pltpu.VMEM((blk, 16), jnp.float32),
            pltpu.VMEM((128, 16), jnp.float32),
            pltpu.VMEM_SHARED((n_pad, 16), jnp.float32),
            pltpu.SemaphoreType.DMA,
            pltpu.SemaphoreType.DMA,
        ],
    )
    def kern(dst_hbm, out_hbm, idx_a, idx_b, ones_v, zero_v, acc, sem_a, sem_b):
        c = lax.axis_index("c")
        s = lax.axis_index("s")
        w = s * _NC + c

        @pl.loop(0, blk)
        def _(r):
            ones_v[r, :] = jnp.ones((16,), jnp.float32)

        @pl.loop(0, 128)
        def _(r):
            zero_v[r, :] = jnp.zeros((16,), jnp.float32)

        @pl.loop(0, rps // 128)
        def _(k2):
            pltpu.sync_copy(zero_v, acc.at[pl.ds(s * rps + k2 * 128, 128)])

        plsc.subcore_barrier()

        base = w * per_w

        @pl.loop(0, steps // 2)
        def _(j):
            i = 2 * j
            pltpu.sync_copy(dst_hbm.at[pl.ds(base + i * blk, blk)], idx_a)
            pltpu.sync_copy(ones_v, acc.at[idx_a], add=True)
            pltpu.sync_copy(dst_hbm.at[pl.ds(base + (i + 1) * blk, blk)], idx_b)
            pltpu.sync_copy(ones_v, acc.at[idx_b], add=True)

        pltpu.sync_copy(dst_hbm.at[pl.ds(base + (steps - 1) * blk, blk)], idx_a)
        pltpu.sync_copy(ones_v, acc.at[idx_a], add=True)

        plsc.subcore_barrier()
        pltpu.sync_copy(acc.at[pl.ds(s * rps, rps)],
                        out_hbm.at[c, pl.ds(s * rps, rps)])

    return kern(dst)


def _sc_agg_colsplit(hsplit, src, dst, n):
    e = src.shape[0]
    dh = hsplit.shape[1]
    per_s = e // _NS
    blk = 80
    steps = per_s // blk
    rps, n_pad = _pad(n)
    zr = 128

    @functools.partial(
        pl.kernel,
        out_type=jax.ShapeDtypeStruct((_NC, n_pad, dh), jnp.float32),
        mesh=_sc_mesh(),
        scratch_types=[
            pltpu.VMEM((blk,), jnp.int32),
            pltpu.VMEM((blk,), jnp.int32),
            pltpu.VMEM((blk,), jnp.int32),
            pltpu.VMEM((blk,), jnp.int32),
            pltpu.VMEM((blk,), jnp.int32),
            pltpu.VMEM((blk,), jnp.int32),
            pltpu.VMEM((blk,), jnp.int32),
            pltpu.VMEM((blk,), jnp.int32),
            pltpu.VMEM((blk,), jnp.int32),
            pltpu.VMEM((blk, dh), jnp.float32),
            pltpu.VMEM((blk, dh), jnp.float32),
            pltpu.VMEM((blk, dh), jnp.float32),
            pltpu.VMEM((zr, dh), jnp.float32),
            pltpu.VMEM_SHARED((n_pad, dh), jnp.float32),
            pltpu.SemaphoreType.DMA,
            pltpu.SemaphoreType.DMA,
            pltpu.SemaphoreType.DMA,
            pltpu.SemaphoreType.DMA,
            pltpu.SemaphoreType.DMA,
            pltpu.SemaphoreType.DMA,
        ],
    )
    def kern(h_hbm, src_hbm, dst_hbm, out_hbm,
             s0, s1, s2, q0, q1, q2, d0, d1, d2, r0, r1, r2, zero_v, acc,
             g0, g1, g2, i0, i1, i2):
        c = lax.axis_index("c")
        s = lax.axis_index("s")
        coff = c * n

        @pl.loop(0, zr)
        def _(r):
            @pl.loop(0, dh // 16)
            def _(g):
                zero_v[r, pl.ds(g * 16, 16)] = jnp.zeros((16,), jnp.float32)

        @pl.loop(0, rps // zr)
        def _(k2):
            pltpu.sync_copy(zero_v, acc.at[pl.ds(s * rps + k2 * zr, zr)])

        plsc.subcore_barrier()

        base = s * per_s
        sbufs, qbufs = (s0, s1, s2), (q0, q1, q2)
        dbufs, rbufs = (d0, d1, d2), (r0, r1, r2)
        gsems, isems = (g0, g1, g2), (i0, i1, i2)

        def issue_idx(k, slot):
            kc = jnp.minimum(k, steps - 1)
            pltpu.async_copy(src_hbm.at[pl.ds(base + kc * blk, blk)],
                             sbufs[slot], isems[slot])
            pltpu.async_copy(dst_hbm.at[pl.ds(base + kc * blk, blk)],
                             dbufs[slot], isems[slot])

        def wait_idx(k, slot):
            kc = jnp.minimum(k, steps - 1)
            pltpu.make_async_copy(src_hbm.at[pl.ds(base + kc * blk, blk)],
                                  sbufs[slot], isems[slot]).wait()
            pltpu.make_async_copy(dst_hbm.at[pl.ds(base + kc * blk, blk)],
                                  dbufs[slot], isems[slot]).wait()

        def adjust(slot):
            sb, qb = sbufs[slot], qbufs[slot]

            @pl.loop(0, blk // 16)
            def _(g):
                qb[pl.ds(g * 16, 16)] = sb[pl.ds(g * 16, 16)] + coff

        def start_gather(slot):
            pltpu.async_copy(h_hbm.at[qbufs[slot]], rbufs[slot], gsems[slot])

        def finish(slot):
            pltpu.make_async_copy(h_hbm.at[qbufs[slot]], rbufs[slot],
                                  gsems[slot]).wait()
            pltpu.sync_copy(rbufs[slot], acc.at[dbufs[slot]], add=True)

        issue_idx(0, 0)
        wait_idx(0, 0)
        adjust(0)
        start_gather(0)
        issue_idx(1, 1)
        wait_idx(1, 1)
        adjust(1)
        start_gather(1)
        issue_idx(2, 2)

        @pl.loop(0, (steps - 2) // 3)
        def _(j):
            k0 = 3 * j
            for kk in range(3):
                k = k0 + kk
                gslot = (kk + 2) % 3
                wait_idx(k + 2, gslot)
                adjust(gslot)
                start_gather(gslot)
                finish(kk)
                issue_idx(k + 3, kk)

        finish((steps - 2) % 3)
        finish((steps - 1) % 3)
        wait_idx(steps - 1, (steps - 3) % 3)

        plsc.subcore_barrier()
        pltpu.sync_copy(acc.at[pl.ds(s * rps, rps)],
                        out_hbm.at[c, pl.ds(s * rps, rps)])

    return kern(hsplit, src, dst)


def _sc_agg_edgesplit(h, src, dst, n):
    e = src.shape[0]
    dh = h.shape[1]
    per_w = e // (_NC * _NS)
    blk = 40
    steps = per_w // blk
    rps, n_pad = _pad(n)
    zr = 128

    @functools.partial(
        pl.kernel,
        out_type=jax.ShapeDtypeStruct((_NC, n_pad, dh), jnp.float32),
        mesh=_sc_mesh(),
        scratch_types=[
            pltpu.VMEM((blk,), jnp.int32),
            pltpu.VMEM((blk,), jnp.int32),
            pltpu.VMEM((blk,), jnp.int32),
            pltpu.VMEM((blk,), jnp.int32),
            pltpu.VMEM((blk,), jnp.int32),
            pltpu.VMEM((blk,), jnp.int32),
            pltpu.VMEM((blk, dh), jnp.float32),
            pltpu.VMEM((blk, dh), jnp.float32),
            pltpu.VMEM((blk, dh), jnp.float32),
            pltpu.VMEM((zr, dh), jnp.float32),
            pltpu.VMEM_SHARED((n_pad, dh), jnp.float32),
            pltpu.SemaphoreType.DMA,
            pltpu.SemaphoreType.DMA,
            pltpu.SemaphoreType.DMA,
            pltpu.SemaphoreType.DMA,
            pltpu.SemaphoreType.DMA,
            pltpu.SemaphoreType.DMA,
        ],
    )
    def kern(h_hbm, src_hbm, dst_hbm, out_hbm,
             s0, s1, s2, d0, d1, d2, r0, r1, r2, zero_v, acc,
             g0, g1, g2, i0, i1, i2):
        c = lax.axis_index("c")
        s = lax.axis_index("s")
        w = s * _NC + c

        @pl.loop(0, zr)
        def _(r):
            @pl.loop(0, dh // 16)
            def _(g):
                zero_v[r, pl.ds(g * 16, 16)] = jnp.zeros((16,), jnp.float32)

        @pl.loop(0, rps // zr)
        def _(k2):
            pltpu.sync_copy(zero_v, acc.at[pl.ds(s * rps + k2 * zr, zr)])

        plsc.subcore_barrier()

        base = w * per_w
        sbufs, dbufs, rbufs = (s0, s1, s2), (d0, d1, d2), (r0, r1, r2)
        gsems, isems = (g0, g1, g2), (i0, i1, i2)

        def issue_idx(k, slot):
            kc = jnp.minimum(k, steps - 1)
            pltpu.async_copy(src_hbm.at[pl.ds(base + kc * blk, blk)],
                             sbufs[slot], isems[slot])
            pltpu.async_copy(dst_hbm.at[pl.ds(base + kc * blk, blk)],
                             dbufs[slot], isems[slot])

        def wait_idx(k, slot):
            kc = jnp.minimum(k, steps - 1)
            pltpu.make_async_copy(src_hbm.at[pl.ds(base + kc * blk, blk)],
                                  sbufs[slot], isems[slot]).wait()
            pltpu.make_async_copy(dst_hbm.at[pl.ds(base + kc * blk, blk)],
                                  dbufs[slot], isems[slot]).wait()

        def start_gather(slot):
            pltpu.async_copy(h_hbm.at[sbufs[slot]], rbufs[slot], gsems[slot])

        def finish(slot):
            pltpu.make_async_copy(h_hbm.at[sbufs[slot]], rbufs[slot],
                                  gsems[slot]).wait()
            pltpu.sync_copy(rbufs[slot], acc.at[dbufs[slot]], add=True)

        issue_idx(0, 0)
        wait_idx(0, 0)
        start_gather(0)
        issue_idx(1, 1)
        wait_idx(1, 1)
        start_gather(1)
        issue_idx(2, 2)

        @pl.loop(0, (steps - 2) // 3)
        def _(j):
            k0 = 3 * j
            for kk in range(3):
                k = k0 + kk
                gslot = (kk + 2) % 3
                wait_idx(k + 2, gslot)
                start_gather(gslot)
                finish(kk)
                issue_idx(k + 3, kk)

        finish((steps - 2) % 3)
        finish((steps - 1) % 3)
        wait_idx(steps - 1, (steps - 3) % 3)

        plsc.subcore_barrier()
        pltpu.sync_copy(acc.at[pl.ds(s * rps, rps)],
                        out_hbm.at[c, pl.ds(s * rps, rps)])

    return kern(h, src, dst)


_BM = 1000


def _tc_first(x, W1, degp):
    n, d_in = x.shape
    d_out = W1.shape[1]
    dh = d_out // 2

    def body(x_ref, w_ref, d_ref, dis_ref, o_ref):
        deg = 1.0 + d_ref[0][:, 0:1] + d_ref[1][:, 0:1]
        dis = lax.rsqrt(deg)
        h = jnp.dot(x_ref[...], w_ref[...], preferred_element_type=jnp.float32)
        hp = h * dis
        dis_ref[...] = dis
        o_ref[0] = hp[:, :dh]
        o_ref[1] = hp[:, dh:]

    return pl.pallas_call(
        body,
        grid=(n // _BM,),
        in_specs=[
            pl.BlockSpec((_BM, d_in), lambda i: (i, 0)),
            pl.BlockSpec((d_in, d_out), lambda i: (0, 0)),
            pl.BlockSpec((2, _BM, 16), lambda i: (0, i, 0)),
        ],
        out_specs=[
            pl.BlockSpec((_BM, 1), lambda i: (i, 0)),
            pl.BlockSpec((2, _BM, dh), lambda i: (0, i, 0)),
        ],
        out_shape=[
            jax.ShapeDtypeStruct((n, 1), jnp.float32),
            jax.ShapeDtypeStruct((2, n, dh), jnp.float32),
        ],
    )(x, W1, degp)


def _tc_mid(aggp, hp, dis, b, W, split_out):
    n = hp.shape[1]
    dh = hp.shape[2]
    d_out = W.shape[1]
    wsplit = W.reshape(2, dh, d_out)
    b2 = b.reshape(2, 1, dh)

    def body(a_ref, h_ref, dis_ref, b_ref, w_ref, o_ref):
        dis = dis_ref[...]
        t = (a_ref[...] + h_ref[...]) * dis[None, :, :] + b_ref[...]
        t = jnp.maximum(t, 0.0)
        h = (jnp.dot(t[0], w_ref[0], preferred_element_type=jnp.float32)
             + jnp.dot(t[1], w_ref[1], preferred_element_type=jnp.float32))
        h = h * dis
        if split_out:
            o_ref[0] = h[:, :d_out // 2]
            o_ref[1] = h[:, d_out // 2:]
        else:
            o_ref[:, :d_out] = h
            o_ref[:, d_out:] = jnp.zeros((h.shape[0], 128 - d_out), jnp.float32)

    if split_out:
        out_spec = pl.BlockSpec((2, _BM, d_out // 2), lambda i: (0, i, 0))
        out_shape = jax.ShapeDtypeStruct((2, n, d_out // 2), jnp.float32)
    else:
        out_spec = pl.BlockSpec((_BM, 128), lambda i: (i, 0))
        out_shape = jax.ShapeDtypeStruct((n, 128), jnp.float32)

    return pl.pallas_call(
        body,
        grid=(n // _BM,),
        in_specs=[
            pl.BlockSpec((2, _BM, dh), lambda i: (0, i, 0)),
            pl.BlockSpec((2, _BM, dh), lambda i: (0, i, 0)),
            pl.BlockSpec((_BM, 1), lambda i: (i, 0)),
            pl.BlockSpec((2, 1, dh), lambda i: (0, 0, 0)),
            pl.BlockSpec((2, dh, d_out), lambda i: (0, 0, 0)),
        ],
        out_specs=out_spec,
        out_shape=out_shape,
    )(aggp, hp, dis, b2, wsplit)


def _tc_final(aggp, h3p, dis, b, d):
    n = h3p.shape[0]
    b2 = b.reshape(1, d)

    def body(a_ref, h_ref, dis_ref, b_ref, o_ref):
        t = a_ref[0] + a_ref[1] + h_ref[...]
        t = t[:, :d] * dis_ref[...] + b_ref[...]
        m = jnp.max(t, axis=1, keepdims=True)
        ex = jnp.exp(t - m)
        lse = jnp.log(jnp.sum(ex, axis=1, keepdims=True))
        o_ref[...] = t - m - lse

    return pl.pallas_call(
        body,
        grid=(n // _BM,),
        in_specs=[
            pl.BlockSpec((2, _BM, 128), lambda i: (0, i, 0)),
            pl.BlockSpec((_BM, 128), lambda i: (i, 0)),
            pl.BlockSpec((_BM, 1), lambda i: (i, 0)),
            pl.BlockSpec((1, d), lambda i: (0, 0)),
        ],
        out_specs=pl.BlockSpec((_BM, d), lambda i: (i, 0)),
        out_shape=jax.ShapeDtypeStruct((n, d), jnp.float32),
    )(aggp, h3p, dis, b2)


def kernel(x, adj_t, W1, b1, W2, b2, W3, b3):
    n = x.shape[0]
    src = adj_t[0]
    dst = adj_t[1]

    degp = _sc_degree(dst, n)
    dis, h1p = _tc_first(x, W1, degp)
    agg1 = _sc_agg_colsplit(h1p.reshape(2 * n, -1), src, dst, n)
    h2p = _tc_mid(agg1, h1p, dis, b1, W2, split_out=True)
    agg2 = _sc_agg_colsplit(h2p.reshape(2 * n, -1), src, dst, n)
    h3p = _tc_mid(agg2, h2p, dis, b2, W3, split_out=False)
    agg3 = _sc_agg_edgesplit(h3p, src, dst, n)
    return _tc_final(agg3, h3p, dis, b3, W3.shape[1])
